# baseline (speedup 1.0000x reference)
import functools

import jax
import jax.numpy as jnp
from jax import lax
from jax.experimental import pallas as pl
from jax.experimental.pallas import tpu as pltpu
from jax.experimental.pallas import tpu_sc as plsc

D = 2048
HALF = D // 2
N_TOK = 8192
F_ROWS = 256
S_ROWS = 1025
S_PAD = 1032

NC = 2
NS = 16
NW = NC * NS
ROWS_PER_W = N_TOK // NW
C = 16
NCH = ROWS_PER_W // C
LANES = 16


def _project_body(fe_ref, se_ref, w_ref, b_ref, fp_ref, sp_ref):
    w = w_ref[...]
    w1 = w[:, :HALF]
    w2 = w[:, HALF:]
    dn = (((1,), (1,)), ((), ()))
    fp_ref[...] = (
        lax.dot_general(fe_ref[...], w1, dn, preferred_element_type=jnp.float32)
        + b_ref[...]
    )
    sp_ref[...] = lax.dot_general(
        se_ref[...], w2, dn, preferred_element_type=jnp.float32
    )


def _project(frame_emb, spatial_emb_pad, w, b2d):
    bn = 256
    grid = (D // bn,)
    return pl.pallas_call(
        _project_body,
        grid=grid,
        in_specs=[
            pl.BlockSpec((F_ROWS, HALF), lambda i: (0, 0)),
            pl.BlockSpec((S_PAD, HALF), lambda i: (0, 0)),
            pl.BlockSpec((bn, D), lambda i: (i, 0)),
            pl.BlockSpec((1, bn), lambda i: (0, i)),
        ],
        out_specs=[
            pl.BlockSpec((F_ROWS, bn), lambda i: (0, i)),
            pl.BlockSpec((S_PAD, bn), lambda i: (0, i)),
        ],
        out_shape=[
            jax.ShapeDtypeStruct((F_ROWS, D), jnp.float32),
            jax.ShapeDtypeStruct((S_PAD, D), jnp.float32),
        ],
    )(frame_emb, spatial_emb_pad, w, b2d)


def _gather_add_body(fp_hbm, sp_hbm, fid_hbm, sid_hbm, out_hbm,
                     fid_v, sid_v, fbuf, sbuf, sem_f, sem_s):
    wid = lax.axis_index("s") * NC + lax.axis_index("c")
    base = wid * ROWS_PER_W
    pltpu.sync_copy(fid_hbm.at[pl.ds(base, ROWS_PER_W)], fid_v)
    pltpu.sync_copy(sid_hbm.at[pl.ds(base, ROWS_PER_W)], sid_v)

    def chunk(ci, carry):
        off = pl.multiple_of(ci * C, C)
        cf = pltpu.async_copy(fp_hbm.at[fid_v.at[pl.ds(off, C)]], fbuf, sem_f)
        cs = pltpu.async_copy(sp_hbm.at[sid_v.at[pl.ds(off, C)]], sbuf, sem_s)
        cf.wait()
        cs.wait()

        def row(r, rc):
            for k in range(D // LANES):
                sl = pl.ds(k * LANES, LANES)
                fbuf[r, sl] = fbuf[r, sl] + sbuf[r, sl]
            return rc

        lax.fori_loop(0, C, row, 0, unroll=False)
        pltpu.sync_copy(fbuf, out_hbm.at[pl.ds(base + off, C)])
        return carry

    lax.fori_loop(0, NCH, chunk, 0, unroll=False)


@functools.partial(
    pl.kernel,
    out_type=jax.ShapeDtypeStruct((N_TOK, D), jnp.float32),
    mesh=plsc.VectorSubcoreMesh(
        core_axis_name="c", subcore_axis_name="s", num_cores=NC, num_subcores=NS
    ),
    scratch_types=[
        pltpu.VMEM((ROWS_PER_W,), jnp.int32),
        pltpu.VMEM((ROWS_PER_W,), jnp.int32),
        pltpu.VMEM((C, D), jnp.float32),
        pltpu.VMEM((C, D), jnp.float32),
        pltpu.SemaphoreType.DMA,
        pltpu.SemaphoreType.DMA,
    ],
)
def _gather_add(fp_hbm, sp_hbm, fid_hbm, sid_hbm, out_hbm,
                fid_v, sid_v, fbuf, sbuf, sem_f, sem_s):
    _gather_add_body(fp_hbm, sp_hbm, fid_hbm, sid_hbm, out_hbm,
                     fid_v, sid_v, fbuf, sbuf, sem_f, sem_s)


def kernel(frame_ids, spatial_ids, frame_emb, spatial_emb, W, b):
    fid = frame_ids.astype(jnp.int32)
    sid = spatial_ids.astype(jnp.int32)
    se_pad = jnp.pad(spatial_emb, ((0, S_PAD - S_ROWS), (0, 0)))
    b2d = b.reshape(1, D)
    fp, sp = _project(frame_emb, se_pad, W, b2d)
    return _gather_add(fp, sp, fid, sid)


# speedup vs baseline: 1.6232x; 1.6232x over previous
import functools

import jax
import jax.numpy as jnp
from jax import lax
from jax.experimental import pallas as pl
from jax.experimental.pallas import tpu as pltpu
from jax.experimental.pallas import tpu_sc as plsc

D = 2048
HALF = D // 2
N_TOK = 8192
F_ROWS = 256
S_ROWS = 1025
S_PAD = 1032

NC = 2
NS = 16
NW = NC * NS
ROWS_PER_W = N_TOK // NW
C = 8
NCH = ROWS_PER_W // C
NPAIR = NCH // 2
LANES = 16


def _project_body(fe_ref, se_ref, w_ref, b_ref, fp_ref, sp_ref):
    w = w_ref[...]
    w1 = w[:, :HALF]
    w2 = w[:, HALF:]
    dn = (((1,), (1,)), ((), ()))
    fp_ref[...] = (
        lax.dot_general(fe_ref[...], w1, dn, preferred_element_type=jnp.float32)
        + b_ref[...]
    )
    sp_ref[...] = lax.dot_general(
        se_ref[...], w2, dn, preferred_element_type=jnp.float32
    )


def _project(frame_emb, spatial_emb_pad, w, b2d):
    bn = 256
    grid = (D // bn,)
    return pl.pallas_call(
        _project_body,
        grid=grid,
        in_specs=[
            pl.BlockSpec((F_ROWS, HALF), lambda i: (0, 0)),
            pl.BlockSpec((S_PAD, HALF), lambda i: (0, 0)),
            pl.BlockSpec((bn, D), lambda i: (i, 0)),
            pl.BlockSpec((1, bn), lambda i: (0, i)),
        ],
        out_specs=[
            pl.BlockSpec((F_ROWS, bn), lambda i: (0, i)),
            pl.BlockSpec((S_PAD, bn), lambda i: (0, i)),
        ],
        out_shape=[
            jax.ShapeDtypeStruct((F_ROWS, D), jnp.float32),
            jax.ShapeDtypeStruct((S_PAD, D), jnp.float32),
        ],
    )(frame_emb, spatial_emb_pad, w, b2d)


def _gather_add_body(fp_hbm, sp_hbm, fid_hbm, sid_hbm, out_hbm,
                     fid_v, sid_v,
                     fbuf0, sbuf0, obuf0, fbuf1, sbuf1, obuf1,
                     gf0, gs0, gf1, gs1, st0, st1):
    wid = lax.axis_index("s") * NC + lax.axis_index("c")
    base = wid * ROWS_PER_W
    pltpu.sync_copy(fid_hbm.at[pl.ds(base, ROWS_PER_W)], fid_v)
    pltpu.sync_copy(sid_hbm.at[pl.ds(base, ROWS_PER_W)], sid_v)

    def issue_gathers(ci, fb, sb, semf, sems):
        off = pl.multiple_of(ci * C, 8)
        pltpu.async_copy(fp_hbm.at[fid_v.at[pl.ds(off, C)]], fb, semf)
        pltpu.async_copy(sp_hbm.at[sid_v.at[pl.ds(off, C)]], sb, sems)

    def wait_gathers(ci, fb, sb, semf, sems):
        off = pl.multiple_of(ci * C, 8)
        pltpu.make_async_copy(fp_hbm.at[fid_v.at[pl.ds(off, C)]], fb, semf).wait()
        pltpu.make_async_copy(sp_hbm.at[sid_v.at[pl.ds(off, C)]], sb, sems).wait()

    def issue_store(ci, ob, sem):
        off = pl.multiple_of(ci * C, 8)
        pltpu.async_copy(ob, out_hbm.at[pl.ds(base + off, C)], sem)

    def wait_store(ob, sem):
        pltpu.make_async_copy(ob, out_hbm.at[pl.ds(base, C)], sem).wait()

    def add_chunk(fb, sb, ob):
        def row(r, rc):
            for k in range(D // LANES):
                sl = pl.ds(k * LANES, LANES)
                ob[r, sl] = fb[r, sl] + sb[r, sl]
            return rc

        lax.fori_loop(0, C, row, 0, unroll=False)

    issue_gathers(0, fbuf0, sbuf0, gf0, gs0)
    issue_gathers(1, fbuf1, sbuf1, gf1, gs1)

    def pair(p, carry):
        a = 2 * p

        def slot(ci, fb, sb, ob, semf, sems, semst):
            wait_gathers(ci, fb, sb, semf, sems)

            @pl.when(p >= 1)
            def _():
                wait_store(ob, semst)

            add_chunk(fb, sb, ob)

            @pl.when(p < NPAIR - 1)
            def _():
                issue_gathers(ci + 2, fb, sb, semf, sems)

            issue_store(ci, ob, semst)

        slot(a, fbuf0, sbuf0, obuf0, gf0, gs0, st0)
        slot(a + 1, fbuf1, sbuf1, obuf1, gf1, gs1, st1)
        return carry

    lax.fori_loop(0, NPAIR, pair, 0, unroll=False)
    wait_store(obuf0, st0)
    wait_store(obuf1, st1)


@functools.partial(
    pl.kernel,
    out_type=jax.ShapeDtypeStruct((N_TOK, D), jnp.float32),
    mesh=plsc.VectorSubcoreMesh(
        core_axis_name="c", subcore_axis_name="s", num_cores=NC, num_subcores=NS
    ),
    scratch_types=[
        pltpu.VMEM((ROWS_PER_W,), jnp.int32),
        pltpu.VMEM((ROWS_PER_W,), jnp.int32),
        pltpu.VMEM((C, D), jnp.float32),
        pltpu.VMEM((C, D), jnp.float32),
        pltpu.VMEM((C, D), jnp.float32),
        pltpu.VMEM((C, D), jnp.float32),
        pltpu.VMEM((C, D), jnp.float32),
        pltpu.VMEM((C, D), jnp.float32),
        pltpu.SemaphoreType.DMA,
        pltpu.SemaphoreType.DMA,
        pltpu.SemaphoreType.DMA,
        pltpu.SemaphoreType.DMA,
        pltpu.SemaphoreType.DMA,
        pltpu.SemaphoreType.DMA,
    ],
)
def _gather_add(fp_hbm, sp_hbm, fid_hbm, sid_hbm, out_hbm,
                fid_v, sid_v,
                fbuf0, sbuf0, obuf0, fbuf1, sbuf1, obuf1,
                gf0, gs0, gf1, gs1, st0, st1):
    _gather_add_body(fp_hbm, sp_hbm, fid_hbm, sid_hbm, out_hbm,
                     fid_v, sid_v,
                     fbuf0, sbuf0, obuf0, fbuf1, sbuf1, obuf1,
                     gf0, gs0, gf1, gs1, st0, st1)


def kernel(frame_ids, spatial_ids, frame_emb, spatial_emb, W, b):
    fid = frame_ids.astype(jnp.int32)
    sid = spatial_ids.astype(jnp.int32)
    se_pad = jnp.pad(spatial_emb, ((0, S_PAD - S_ROWS), (0, 0)))
    b2d = b.reshape(1, D)
    fp, sp = _project(frame_emb, se_pad, W, b2d)
    return _gather_add(fp, sp, fid, sid)
